# baseline (speedup 1.0000x reference)
import functools

import jax
import jax.numpy as jnp
from jax import lax
from jax.experimental import pallas as pl
from jax.experimental.pallas import tpu as pltpu
from jax.experimental.pallas import tpu_sc as plsc

B = 16384
NF = 26
V = 100000
D = 64
NG = 10
L = 16

NS = 16
BH = B // 2
BW = BH // NS
BC = 32
NCHUNK = BW // BC


def _make_body(base):
    def _body(fidx, gw, tab, emb, out, idx_v, s_v, w_v, e_v, gsem):
        wid = lax.axis_index("s")
        pltpu.sync_copy(emb, e_v)

        def chunk(c, carry):
            b0 = base + wid * BW + c * BC
            pltpu.sync_copy(fidx.at[pl.ds(b0, BC), :], idx_v)
            cps = [
                pltpu.async_copy(
                    tab.at[idx_v.at[b]], s_v.at[b, pl.ds(0, NF), :], gsem
                )
                for b in range(BC)
            ]
            pltpu.sync_copy(gw.at[pl.ds(b0, BC), :], w_v)
            for b in range(BC):
                wv = w_v[b, :]
                ws = [wv[g] for g in range(NG)]
                q = ws[0]
                for g in range(1, NG):
                    q = q + ws[g]
                qi = 1.0 / jnp.broadcast_to(q, (L,))
                for k in range(D // L):
                    acc = ws[0] * e_v[0, pl.ds(k * L, L)]
                    for g in range(1, NG):
                        acc = acc + ws[g] * e_v[g, pl.ds(k * L, L)]
                    s_v[b, NF, pl.ds(k * L, L)] = acc * qi
            for cp in cps:
                cp.wait()
            pltpu.sync_copy(s_v, out.at[pl.ds(b0 - base, BC), :, :])
            return carry

        lax.fori_loop(0, NCHUNK, chunk, 0)

    return _body


@jax.jit
def _embed(fidx, gw, tab, emb):
    outs = []
    for half in range(2):
        mesh = plsc.VectorSubcoreMesh(
            core_axis_name="c", subcore_axis_name="s", num_cores=1
        )
        kfn = functools.partial(
            pl.kernel,
            mesh=mesh,
            compiler_params=pltpu.CompilerParams(use_tc_tiling_on_sc=False),
            out_type=jax.ShapeDtypeStruct((BH, NF + 1, D), jnp.float32),
            scratch_types=[
                pltpu.VMEM((BC, NF), jnp.int32),
                pltpu.VMEM((BC, NF + 1, D), jnp.float32),
                pltpu.VMEM((BC, L), jnp.float32),
                pltpu.VMEM((NG, D), jnp.float32),
                pltpu.SemaphoreType.DMA,
            ],
            name=f"embed_half{half}",
        )(_make_body(half * BH))
        outs.append(kfn(fidx, gw, tab, emb))
    return jnp.concatenate(outs, axis=0)


def kernel(x, tables, genre_embed):
    fidx = x[:, :NF] + (jnp.arange(NF, dtype=jnp.int32) * V)[None, :]
    gw = jnp.pad(x[:, NF:].astype(jnp.float32), ((0, 0), (0, L - NG)))
    tab = tables.reshape(NF * V, D)
    return _embed(fidx, gw, tab, genre_embed)


# speedup vs baseline: 1.0018x; 1.0018x over previous
import functools

import jax
import jax.numpy as jnp
from jax import lax
from jax.experimental import pallas as pl
from jax.experimental.pallas import tpu as pltpu
from jax.experimental.pallas import tpu_sc as plsc

B = 16384
NF = 26
V = 100000
D = 64
NG = 10
L = 16

NS = 16
BH = B // 2
BW = BH // NS
BC = 32
NCHUNK = BW // BC


def _make_body(base):
    def _body(fidx, gw, tab, emb, out, idx_v, s_v, w_v, e_v, gsem):
        wid = lax.axis_index("s")
        pltpu.sync_copy(emb, e_v)

        def chunk(c, carry):
            b0 = base + wid * BW + c * BC
            pltpu.sync_copy(fidx.at[pl.ds(b0, BC), :], idx_v)
            cps = [
                pltpu.async_copy(
                    tab.at[idx_v.at[b]], s_v.at[b, pl.ds(0, NF), :], gsem
                )
                for b in range(BC)
            ]
            pltpu.sync_copy(gw.at[pl.ds(b0, BC), :], w_v)
            for b in range(BC):
                wv = w_v[b, :]
                ws = [wv[g] for g in range(NG)]
                q = ws[0]
                for g in range(1, NG):
                    q = q + ws[g]
                qi = 1.0 / jnp.broadcast_to(q, (L,))
                for k in range(D // L):
                    acc = ws[0] * e_v[0, pl.ds(k * L, L)]
                    for g in range(1, NG):
                        acc = acc + ws[g] * e_v[g, pl.ds(k * L, L)]
                    s_v[b, NF, pl.ds(k * L, L)] = acc * qi
            for cp in cps:
                cp.wait()
            pltpu.sync_copy(s_v, out.at[pl.ds(b0 - base, BC), :, :])
            return carry

        lax.fori_loop(0, NCHUNK, chunk, 0)

    return _body


@jax.jit
def _embed(fidx, gw, tab, emb):
    outs = []
    for half in range(2):
        mesh = plsc.VectorSubcoreMesh(
            core_axis_name="c", subcore_axis_name="s", num_cores=1
        )
        kfn = functools.partial(
            pl.kernel,
            mesh=mesh,
            compiler_params=pltpu.CompilerParams(use_tc_tiling_on_sc=False, has_side_effects=False),
            out_type=jax.ShapeDtypeStruct((BH, NF + 1, D), jnp.float32),
            scratch_types=[
                pltpu.VMEM((BC, NF), jnp.int32),
                pltpu.VMEM((BC, NF + 1, D), jnp.float32),
                pltpu.VMEM((BC, L), jnp.float32),
                pltpu.VMEM((NG, D), jnp.float32),
                pltpu.SemaphoreType.DMA,
            ],
            name=f"embed_half{half}",
        )(_make_body(half * BH))
        outs.append(kfn(fidx, gw, tab, emb))
    return jnp.concatenate(outs, axis=0)


def kernel(x, tables, genre_embed):
    fidx = x[:, :NF] + (jnp.arange(NF, dtype=jnp.int32) * V)[None, :]
    gw = jnp.pad(x[:, NF:].astype(jnp.float32), ((0, 0), (0, L - NG)))
    tab = tables.reshape(NF * V, D)
    return _embed(fidx, gw, tab, genre_embed)


# speedup vs baseline: 1.2247x; 1.2224x over previous
import functools

import jax
import jax.numpy as jnp
from jax import lax
from jax.experimental import pallas as pl
from jax.experimental.pallas import tpu as pltpu
from jax.experimental.pallas import tpu_sc as plsc

B = 16384
NF = 26
V = 100000
D = 64
DP = 128
NG = 10
L = 16

NC = 2
NS = 16
NW = NC * NS
BW = B // NW
BC = 16
NCHUNK = BW // BC


def _body(fidx, gw, tab, emb, out, idx_v, s_v, w_v, e_v, gsem):
    wid = lax.axis_index("s") * NC + lax.axis_index("c")
    pltpu.sync_copy(emb, e_v)

    def chunk(c, carry):
        b0 = wid * BW + c * BC
        pltpu.sync_copy(fidx.at[pl.ds(b0, BC), :], idx_v)
        cps = [
            pltpu.async_copy(
                tab.at[idx_v.at[b]], s_v.at[b, pl.ds(0, NF), :], gsem
            )
            for b in range(BC)
        ]
        pltpu.sync_copy(gw.at[pl.ds(b0, BC), :], w_v)
        for b in range(BC):
            wv = w_v[b, :]
            ws = [wv[g] for g in range(NG)]
            q = ws[0]
            for g in range(1, NG):
                q = q + ws[g]
            qi = 1.0 / jnp.broadcast_to(q, (L,))
            for k in range(D // L):
                acc = ws[0] * e_v[0, pl.ds(k * L, L)]
                for g in range(1, NG):
                    acc = acc + ws[g] * e_v[g, pl.ds(k * L, L)]
                s_v[b, NF, pl.ds(k * L, L)] = acc * qi
        for cp in cps:
            cp.wait()
        pltpu.sync_copy(s_v, out.at[pl.ds(b0, BC), :, :])
        return carry

    lax.fori_loop(0, NCHUNK, chunk, 0)


@jax.jit
def _embed(fidx, gw, tab, emb):
    mesh = plsc.VectorSubcoreMesh(core_axis_name="c", subcore_axis_name="s")
    kfn = functools.partial(
        pl.kernel,
        mesh=mesh,
        out_type=jax.ShapeDtypeStruct((B, NF + 1, DP), jnp.float32),
        scratch_types=[
            pltpu.VMEM((BC, NF), jnp.int32),
            pltpu.VMEM((BC, NF + 1, DP), jnp.float32),
            pltpu.VMEM((BC, L), jnp.float32),
            pltpu.VMEM((NG, D), jnp.float32),
            pltpu.SemaphoreType.DMA,
        ],
    )(_body)
    return kfn(fidx, gw, tab, emb)[:, :, :D]


def kernel(x, tables, genre_embed):
    fidx = x[:, :NF] + (jnp.arange(NF, dtype=jnp.int32) * V)[None, :]
    gw = jnp.pad(x[:, NF:].astype(jnp.float32), ((0, 0), (0, L - NG)))
    tab = jnp.pad(tables.reshape(NF * V, D), ((0, 0), (0, DP - D)))
    return _embed(fidx, gw, tab, genre_embed)
